# Initial kernel scaffold; baseline (speedup 1.0000x reference)
import jax
import jax.numpy as jnp
from jax.experimental import pallas as pl


def kernel(pattern_table, row_pos, kind_table, board, active_kind, next_kind):
    raise NotImplementedError("write your pallas kernel here")


# speedup vs baseline: 1.7291x; 1.7291x over previous
import functools

import numpy as np
import jax
import jax.numpy as jnp
from jax import lax
from jax.experimental import pallas as pl
from jax.experimental.pallas import tpu as pltpu
from jax.experimental.pallas import tpu_sc as plsc

B = 4096
H = 20
W = 10
D = 128
NK = 7
NPAT = 1 << W
TOK = H + 2
KIND_BASE = H * NPAT
TBL_ROWS = KIND_BASE + 8

NC = 2
NS = 16
NW = NC * NS
ROWS = B * TOK
RPW = ROWS // NW
CHUNK = 128
NCH = RPW // CHUNK


def _packing_matrix() -> np.ndarray:
    pm = np.zeros((H * W + 2, TOK), np.float32)
    for h in range(H):
        for w in range(W):
            pm[h * W + w, h] = float(1 << w)
    pm[H * W, H] = 1.0
    pm[H * W + 1, H + 1] = 1.0
    return pm


def _row_offsets() -> np.ndarray:
    offs = [h * NPAT for h in range(H)] + [KIND_BASE, KIND_BASE]
    return np.asarray(offs, np.int32).reshape(1, TOK)


def _prep_body(pmat_ref, offs_ref, pattern_ref, row_pos_ref, kind_ref,
               boardx_ref, table_ref, ids_ref):
    for h in range(H):
        table_ref[pl.ds(h * NPAT, NPAT), :] = (
            pattern_ref[...] + row_pos_ref[pl.ds(h, 1), :])
    table_ref[pl.ds(KIND_BASE, 8), :] = kind_ref[...]
    idsf = jnp.dot(boardx_ref[...], pmat_ref[...],
                   preferred_element_type=jnp.float32)
    ids_ref[...] = idsf.astype(jnp.int32) + offs_ref[...]


_prep = pl.pallas_call(
    _prep_body,
    out_shape=(
        jax.ShapeDtypeStruct((TBL_ROWS, D), jnp.float32),
        jax.ShapeDtypeStruct((B, TOK), jnp.int32),
    ),
)


def _gather_body(table_hbm, ids_hbm, out_hbm, idx_v, rows_v, gsem, ssem):
    wid = lax.axis_index("s") * NC + lax.axis_index("c")
    base = wid * RPW
    pltpu.sync_copy(ids_hbm.at[pl.ds(base, RPW)], idx_v)

    def start_gather(ch):
        return pltpu.async_copy(
            table_hbm.at[idx_v.at[pl.ds(ch * CHUNK, CHUNK)]],
            rows_v.at[ch % 2], gsem)

    def start_scatter(ch):
        return pltpu.async_copy(
            rows_v.at[ch % 2],
            out_hbm.at[pl.ds(base + ch * CHUNK, CHUNK)], ssem)

    gd = start_gather(0)
    sd = [None, None]
    for ch in range(NCH):
        nxt_gd = None
        if ch + 1 < NCH:
            if sd[(ch + 1) % 2] is not None:
                sd[(ch + 1) % 2].wait()
            nxt_gd = start_gather(ch + 1)
        gd.wait()
        sd[ch % 2] = start_scatter(ch)
        gd = nxt_gd
    sd[(NCH - 2) % 2].wait()
    sd[(NCH - 1) % 2].wait()


_gather = functools.partial(
    pl.kernel,
    out_type=jax.ShapeDtypeStruct((ROWS, D), jnp.float32),
    mesh=plsc.VectorSubcoreMesh(core_axis_name="c", subcore_axis_name="s",
                                num_cores=NC, num_subcores=NS),
    scratch_types=[
        pltpu.VMEM((RPW,), jnp.int32),
        pltpu.VMEM((2, CHUNK, D), jnp.float32),
        pltpu.SemaphoreType.DMA,
        pltpu.SemaphoreType.DMA,
    ],
)(_gather_body)


def kernel(pattern_table, row_pos, kind_table, board, active_kind, next_kind):
    boardf = board.reshape(B, H * W).astype(jnp.float32)
    akf = active_kind.astype(jnp.float32).reshape(B, 1)
    nkf = next_kind.astype(jnp.float32).reshape(B, 1)
    boardx = jnp.concatenate([boardf, akf, nkf], axis=1)
    kind_pad = jnp.concatenate(
        [kind_table, jnp.zeros((8 - NK, D), jnp.float32)], axis=0)
    pmat = jnp.asarray(_packing_matrix())
    offs = jnp.asarray(_row_offsets())
    table, ids2d = _prep(pmat, offs, pattern_table, row_pos, kind_pad, boardx)
    out = _gather(table, ids2d.reshape(ROWS))
    return out.reshape(B, TOK, D)
